# baseline (speedup 1.0000x reference)
import functools

import jax
import jax.numpy as jnp
from jax import lax
from jax.experimental import pallas as pl
from jax.experimental.pallas import tpu as pltpu
from jax.experimental.pallas import tpu_sc as plsc

EMBED_DIM = 32
BATCH = 16384
FIELDS = 26
NUM_CORES = 2
NUM_SUBCORES = 16
NW = NUM_CORES * NUM_SUBCORES
TASK_B = 1024
CT_PER_TASK = TASK_B // 128
NTASK = (BATCH // TASK_B) * FIELDS
TASKS_PER_W = NTASK // NW
CTC_PER_F = BATCH // TASK_B

_mesh = plsc.VectorSubcoreMesh(core_axis_name="c", subcore_axis_name="s")


@functools.partial(
    pl.kernel,
    mesh=_mesh,
    out_type=jax.ShapeDtypeStruct((FIELDS, EMBED_DIM // 8, BATCH // 128, 8, 128),
                                  jnp.float32),
    scratch_types=[
        pltpu.VMEM((TASK_B,), jnp.int32),
        pltpu.VMEM((TASK_B, EMBED_DIM), jnp.float32),
        pltpu.VMEM((CT_PER_TASK, 8, 128), jnp.float32),
        pltpu.SemaphoreType.DMA,
    ],
    compiler_params=pltpu.CompilerParams(use_tc_tiling_on_sc=False,
                                         needs_layout_passes=False),
)
def _gather_kernel(idx_hbm, table_hbm, out_hbm, idx_v, gbuf, stage, gsem):
    wid = lax.axis_index("s") * NUM_CORES + lax.axis_index("c")
    iota = lax.iota(jnp.int32, 16)

    def task_body(tl, _):
        t = wid * TASKS_PER_W + tl
        f = t // CTC_PER_F
        ctc = t % CTC_PER_F
        b0 = ctc * TASK_B
        ct0 = ctc * CT_PER_TASK
        pltpu.sync_copy(idx_hbm.at[f, pl.ds(b0, TASK_B)], idx_v)
        pltpu.async_copy(table_hbm.at[idx_v], gbuf, gsem).wait()

        def blk_body(blk, _):
            def ct_body(ctl, _):
                rowbase = ctl * 128
                for r in range(8):
                    col = jnp.zeros((16,), jnp.int32) + (blk * 8 + r)
                    for c0 in range(0, 128, 16):
                        vals = plsc.load_gather(
                            gbuf, [rowbase + c0 + iota, col])
                        stage[ctl, r, pl.ds(c0, 16)] = vals
                return 0
            lax.fori_loop(0, CT_PER_TASK, ct_body, 0)
            pltpu.sync_copy(stage, out_hbm.at[f, blk, pl.ds(ct0, CT_PER_TASK)])
            return 0
        lax.fori_loop(0, EMBED_DIM // 8, blk_body, 0)
        return 0

    lax.fori_loop(0, TASKS_PER_W, task_body, 0)


def kernel(pokemon_state, table):
    idx_t = pokemon_state.T.astype(jnp.int32)
    out5 = _gather_kernel(idx_t, table)
    return out5.transpose(2, 4, 0, 1, 3).reshape(BATCH, FIELDS, EMBED_DIM)


# speedup vs baseline: 1.1280x; 1.1280x over previous
import functools

import jax
import jax.numpy as jnp
from jax import lax
from jax.experimental import pallas as pl
from jax.experimental.pallas import tpu as pltpu
from jax.experimental.pallas import tpu_sc as plsc

EMBED_DIM = 32
BATCH = 16384
FIELDS = 26
B = BATCH * FIELDS
NUM_CORES = 2
NUM_SUBCORES = 16
NW = NUM_CORES * NUM_SUBCORES
B_PER_W = B // NW
CHUNK = 1664
NCHUNK = B_PER_W // CHUNK

_mesh = plsc.VectorSubcoreMesh(core_axis_name="c", subcore_axis_name="s")


@functools.partial(
    pl.kernel,
    mesh=_mesh,
    out_type=jax.ShapeDtypeStruct((B, EMBED_DIM), jnp.float32),
    scratch_types=[
        pltpu.VMEM((B_PER_W,), jnp.int32),
        pltpu.VMEM((2, CHUNK, EMBED_DIM), jnp.float32),
        pltpu.SemaphoreType.DMA,
        pltpu.SemaphoreType.DMA,
        pltpu.SemaphoreType.DMA,
        pltpu.SemaphoreType.DMA,
    ],
    compiler_params=pltpu.CompilerParams(use_tc_tiling_on_sc=False),
)
def _gather_kernel(idx_hbm, table_hbm, out_hbm, idx_v, rows_v,
                   gsem0, gsem1, wsem0, wsem1):
    wid = lax.axis_index("s") * NUM_CORES + lax.axis_index("c")
    base = wid * B_PER_W
    pltpu.sync_copy(idx_hbm.at[wid], idx_v)

    def scale_body(i, _):
        idx_v[pl.ds(i * 16, 16)] = idx_v[pl.ds(i * 16, 16)] * 4
        return 0
    lax.fori_loop(0, B_PER_W // 16, scale_body, 0)

    gsems = [gsem0, gsem1]
    wsems = [wsem0, wsem1]
    gathers = [None, None]
    writes = [None, None]
    gathers[0] = pltpu.async_copy(
        table_hbm.at[idx_v.at[pl.ds(0, CHUNK)]], rows_v.at[0], gsems[0])
    for i in range(NCHUNK):
        b = i % 2
        nb = (i + 1) % 2
        if i + 1 < NCHUNK:
            if writes[nb] is not None:
                writes[nb].wait()
            gathers[nb] = pltpu.async_copy(
                table_hbm.at[idx_v.at[pl.ds((i + 1) * CHUNK, CHUNK)]],
                rows_v.at[nb], gsems[nb])
        gathers[b].wait()
        writes[b] = pltpu.async_copy(rows_v.at[b],
                                     out_hbm.at[pl.ds(base + i * CHUNK, CHUNK)],
                                     wsems[b])
    writes[0].wait()
    writes[1].wait()


def kernel(pokemon_state, table):
    idx = pokemon_state.reshape(NW, B_PER_W).astype(jnp.int32)
    table4 = jnp.pad(table, ((0, 0), (0, 96))).reshape(4 * 1000000, EMBED_DIM)
    out = _gather_kernel(idx, table4)
    return out.reshape(BATCH, FIELDS, EMBED_DIM)


# speedup vs baseline: 1.4452x; 1.2813x over previous
import functools

import jax
import jax.numpy as jnp
from jax import lax
from jax.experimental import pallas as pl
from jax.experimental.pallas import tpu as pltpu
from jax.experimental.pallas import tpu_sc as plsc

EMBED_DIM = 32
BATCH = 16384
FIELDS = 26
NUM_CORES = 2
NUM_SUBCORES = 16
NW = NUM_CORES * NUM_SUBCORES
TASK_B = 1024
CT_PER_TASK = TASK_B // 128
NTASK = (BATCH // TASK_B) * FIELDS
TASKS_PER_W = NTASK // NW
CTC_PER_F = BATCH // TASK_B
SPAD = 131

_mesh = plsc.VectorSubcoreMesh(core_axis_name="c", subcore_axis_name="s")


@functools.partial(
    pl.kernel,
    mesh=_mesh,
    out_type=jax.ShapeDtypeStruct((FIELDS, EMBED_DIM // 8, BATCH // 128, 8, 128),
                                  jnp.float32),
    scratch_types=[
        pltpu.VMEM((2, TASK_B), jnp.int32),
        pltpu.VMEM((2, TASK_B, EMBED_DIM), jnp.float32),
        pltpu.VMEM((CT_PER_TASK, 8, SPAD), jnp.float32),
        pltpu.VMEM((CT_PER_TASK, 8, SPAD), jnp.float32),
        pltpu.VMEM((CT_PER_TASK, 8, SPAD), jnp.float32),
        pltpu.VMEM((CT_PER_TASK, 8, SPAD), jnp.float32),
        pltpu.SemaphoreType.DMA,
        pltpu.SemaphoreType.DMA,
        pltpu.SemaphoreType.DMA,
    ],
    compiler_params=pltpu.CompilerParams(use_tc_tiling_on_sc=False,
                                         needs_layout_passes=False),
)
def _gather_kernel(idx_hbm, table_hbm, out_hbm, idx_v, gbuf,
                   st0, st1, st2, st3, gsem0, gsem1, wsem):
    wid = lax.axis_index("s") * NUM_CORES + lax.axis_index("c")
    iota = lax.iota(jnp.int32, 16)
    rvec = iota & 7
    m_lo = iota < 8
    m_hi = iota >= 8
    gsems = [gsem0, gsem1]
    stages = [st0, st1, st2, st3]

    def load_idx_and_gather(tl, buf):
        t = wid * TASKS_PER_W + tl
        f = t // CTC_PER_F
        b0 = (t % CTC_PER_F) * TASK_B
        pltpu.sync_copy(idx_hbm.at[f, pl.ds(b0, TASK_B)], idx_v.at[buf])

        def scale_body(i, _):
            idx_v[buf, pl.ds(i * 16, 16)] = idx_v[buf, pl.ds(i * 16, 16)] * 4
            return 0
        lax.fori_loop(0, TASK_B // 16, scale_body, 0)
        return pltpu.async_copy(table_hbm.at[idx_v.at[buf]],
                                gbuf.at[buf], gsems[buf])

    gathers = [None, None]
    gathers[0] = load_idx_and_gather(0, 0)
    for tl in range(TASKS_PER_W):
        buf = tl % 2
        nbuf = (tl + 1) % 2
        gathers[buf].wait()
        if tl + 1 < TASKS_PER_W:
            gathers[nbuf] = load_idx_and_gather(tl + 1, nbuf)

        t = wid * TASKS_PER_W + tl
        f = t // CTC_PER_F
        ct0 = (t % CTC_PER_F) * CT_PER_TASK

        def ct_body(ctl, _):
            d0 = jnp.zeros((16,), jnp.int32) + ctl

            def c16_body(c16, _):
                cbase = c16 * 16
                b_base = ctl * 128 + cbase
                for bi in range(16):
                    d2 = jnp.zeros((16,), jnp.int32) + (cbase + bi)
                    v0 = gbuf[buf, b_base + bi, pl.ds(0, 16)]
                    v1 = gbuf[buf, b_base + bi, pl.ds(16, 16)]
                    plsc.store_scatter(st0, [d0, rvec, d2], v0, mask=m_lo)
                    plsc.store_scatter(st1, [d0, rvec, d2], v0, mask=m_hi)
                    plsc.store_scatter(st2, [d0, rvec, d2], v1, mask=m_lo)
                    plsc.store_scatter(st3, [d0, rvec, d2], v1, mask=m_hi)
                return 0
            lax.fori_loop(0, 8, c16_body, 0)
            return 0
        lax.fori_loop(0, CT_PER_TASK, ct_body, 0)

        for blk in range(4):
            pltpu.async_copy(stages[blk].at[:, :, pl.ds(0, 128)],
                             out_hbm.at[f, blk, pl.ds(ct0, CT_PER_TASK)],
                             wsem).wait()


def kernel(pokemon_state, table):
    idx_t = pokemon_state.T.astype(jnp.int32)
    table4 = jnp.pad(table, ((0, 0), (0, 96))).reshape(4 * 1000000, EMBED_DIM)
    out5 = _gather_kernel(idx_t, table4)
    return out5.transpose(2, 4, 0, 1, 3).reshape(BATCH, FIELDS, EMBED_DIM)
